# baseline (speedup 1.0000x reference)
import functools

import jax
import jax.numpy as jnp
from jax import lax
from jax.experimental import pallas as pl
from jax.experimental.pallas import tpu as pltpu
from jax.experimental.pallas import tpu_sc as plsc

_E = 16
_BLK = 2048


def _router_body(x_ref, wt_ref, s_out_ref, i_out_ref):
    logits = jnp.dot(x_ref[...], wt_ref[...], preferred_element_type=jnp.float32)
    iota_e = lax.broadcasted_iota(jnp.int32, (_BLK, _E), 1)
    m1 = jnp.max(logits, axis=1, keepdims=True)
    i1 = jnp.min(jnp.where(logits == m1, iota_e, _E), axis=1, keepdims=True)
    masked = jnp.where(iota_e == i1, -jnp.inf, logits)
    m2 = jnp.max(masked, axis=1, keepdims=True)
    i2 = jnp.min(jnp.where(masked == m2, iota_e, _E), axis=1, keepdims=True)
    s_out_ref[...] = jnp.transpose(jnp.concatenate([m1, m2], axis=1))
    i_out_ref[...] = jnp.transpose(jnp.concatenate([i1, i2], axis=1))


def _make_sc_softmax(n_tok):
    info = plsc.get_sparse_core_info()
    nc, ns, L = info.num_cores, info.num_subcores, info.num_lanes
    nw = nc * ns
    per_w = n_tok // nw
    mesh = plsc.VectorSubcoreMesh(core_axis_name="c", subcore_axis_name="s")

    @functools.partial(
        pl.kernel,
        mesh=mesh,
        out_type=jax.ShapeDtypeStruct((2, n_tok), jnp.float32),
        scratch_types=[pltpu.VMEM((2, per_w), jnp.float32)],
    )
    def sc_softmax(scores_hbm, out_hbm, scr):
        wid = lax.axis_index("s") * nc + lax.axis_index("c")
        base = wid * per_w
        pltpu.sync_copy(scores_hbm.at[:, pl.ds(base, per_w)], scr)
        for j in range(per_w // L):
            sl = pl.ds(j * L, L)
            m1 = scr[0, sl]
            m2 = scr[1, sl]
            e2 = jnp.exp(m2 - m1)
            w1 = 1.0 / (1.0 + e2)
            scr[0, sl] = w1
            scr[1, sl] = 1.0 - w1
        pltpu.sync_copy(scr, out_hbm.at[:, pl.ds(base, per_w)])

    return sc_softmax


@jax.jit
def kernel(x, W):
    B, T, D = x.shape
    n_tok = B * T
    xf = x.reshape(n_tok, D)
    wt = W.T

    grid = (n_tok // _BLK,)
    s_out, i_out = pl.pallas_call(
        _router_body,
        grid=grid,
        in_specs=[
            pl.BlockSpec((_BLK, D), lambda i: (i, 0)),
            pl.BlockSpec((D, _E), lambda i: (0, 0)),
        ],
        out_specs=[
            pl.BlockSpec((2, _BLK), lambda i: (0, i)),
            pl.BlockSpec((2, _BLK), lambda i: (0, i)),
        ],
        out_shape=[
            jax.ShapeDtypeStruct((2, n_tok), jnp.float32),
            jax.ShapeDtypeStruct((2, n_tok), jnp.int32),
        ],
        compiler_params=pltpu.CompilerParams(
            dimension_semantics=("parallel",),
        ),
    )(xf, wt)

    w_out = _make_sc_softmax(n_tok)(s_out)
    return (w_out.T.reshape(B, T, 2), i_out.T.reshape(B, T, 2))


# speedup vs baseline: 1.3449x; 1.3449x over previous
import jax
import jax.numpy as jnp
from jax import lax
from jax.experimental import pallas as pl
from jax.experimental.pallas import tpu as pltpu

_E = 16
_BLK = 2048


def _router_body(x_ref, wt_ref, w_out_ref, i_out_ref):
    logits = jnp.dot(x_ref[...], wt_ref[...], preferred_element_type=jnp.float32)
    iota_e = lax.broadcasted_iota(jnp.int32, (_BLK, _E), 1)
    m1 = jnp.max(logits, axis=1, keepdims=True)
    i1 = jnp.min(jnp.where(logits == m1, iota_e, _E), axis=1, keepdims=True)
    masked = jnp.where(iota_e == i1, -jnp.inf, logits)
    m2 = jnp.max(masked, axis=1, keepdims=True)
    i2 = jnp.min(jnp.where(masked == m2, iota_e, _E), axis=1, keepdims=True)
    e2 = jnp.exp(m2 - m1)
    w1 = 1.0 / (1.0 + e2)
    w2 = e2 * w1
    w_out_ref[...] = jnp.transpose(jnp.concatenate([w1, w2], axis=1))
    i_out_ref[...] = jnp.transpose(jnp.concatenate([i1, i2], axis=1))


@jax.jit
def kernel(x, W):
    B, T, D = x.shape
    n_tok = B * T
    xf = x.reshape(n_tok, D)
    wt = W.T

    grid = (n_tok // _BLK,)
    w_out, i_out = pl.pallas_call(
        _router_body,
        grid=grid,
        in_specs=[
            pl.BlockSpec((_BLK, D), lambda i: (i, 0)),
            pl.BlockSpec((D, _E), lambda i: (0, 0)),
        ],
        out_specs=[
            pl.BlockSpec((2, _BLK), lambda i: (0, i)),
            pl.BlockSpec((2, _BLK), lambda i: (0, i)),
        ],
        out_shape=[
            jax.ShapeDtypeStruct((2, n_tok), jnp.float32),
            jax.ShapeDtypeStruct((2, n_tok), jnp.int32),
        ],
        compiler_params=pltpu.CompilerParams(
            dimension_semantics=("parallel",),
        ),
    )(xf, wt)

    return (w_out.T.reshape(B, T, 2), i_out.T.reshape(B, T, 2))
